# baseline (speedup 1.0000x reference)
import functools

import jax
import jax.numpy as jnp
from jax import lax
from jax.experimental import pallas as pl
from jax.experimental.pallas import tpu as pltpu
from jax.experimental.pallas import tpu_sc as plsc

NE, NR, D, B = 100000, 1000, 64, 512

_NC, _NS = 2, 16
_NW = _NC * _NS
_BPW = B // _NW
_PCH = 8192


def _pack_body(in_ref, out_ref):
    eye = (lax.broadcasted_iota(jnp.int32, (D, D), 0)
           == lax.broadcasted_iota(jnp.int32, (D, D), 1)).astype(jnp.float32)
    out_ref[...] = lax.dot_general(in_ref[...], eye, (((0,), (0,)), ((), ())),
                                   preferred_element_type=jnp.float32)


def _pack(table_t):
    n = table_t.shape[1]
    grid = (n + _PCH - 1) // _PCH
    return pl.pallas_call(
        _pack_body,
        grid=(grid,),
        in_specs=[pl.BlockSpec((D, _PCH), lambda j: (0, j))],
        out_specs=pl.BlockSpec((_PCH, D), lambda j: (j, 0)),
        out_shape=jax.ShapeDtypeStruct((n, D), jnp.float32),
    )(table_t)


def _gather_body_a(h_hbm, r_hbm, t_hbm, ent_emb, rel_emb, rel_proj,
                   he_out, te_out, re_out, pr_out,
                   idx_v, row_bufs, sem):
    wid = lax.axis_index("s") * _NC + lax.axis_index("c")
    base = wid * _BPW
    sl = pl.ds(base, _BPW)

    pltpu.sync_copy(h_hbm.at[sl], idx_v.at[0])
    pltpu.sync_copy(r_hbm.at[sl], idx_v.at[1])
    pltpu.sync_copy(t_hbm.at[sl], idx_v.at[2])
    hv = idx_v[0]
    rv = idx_v[1]
    tv = idx_v[2]

    jobs = ((hv, ent_emb, 0), (tv, ent_emb, 1), (rv, rel_emb, 2),
            (rv, rel_proj, 3))
    handles = []
    for vec, table, buf in jobs:
        for k in range(_BPW):
            ik = vec[k]
            handles.append(pltpu.async_copy(
                table.at[pl.ds(ik, 1)], row_bufs.at[buf, pl.ds(k, 1)], sem))
    for hnd in handles:
        hnd.wait()

    for buf, out in enumerate((he_out, te_out, re_out, pr_out)):
        pltpu.sync_copy(row_bufs.at[buf], out.at[sl])


def _gather_body_b(h_hbm, t_hbm, ent_proj,
                   ph_out, pt_out,
                   idx_v, row_bufs, sem):
    wid = lax.axis_index("s") * _NC + lax.axis_index("c")
    base = wid * _BPW
    sl = pl.ds(base, _BPW)

    pltpu.sync_copy(h_hbm.at[sl], idx_v.at[0])
    pltpu.sync_copy(t_hbm.at[sl], idx_v.at[1])
    hv = idx_v[0]
    tv = idx_v[1]

    handles = []
    for vec, buf in ((hv, 0), (tv, 1)):
        for k in range(_BPW):
            ik = vec[k]
            handles.append(pltpu.async_copy(
                ent_proj.at[pl.ds(ik, 1)], row_bufs.at[buf, pl.ds(k, 1)], sem))
    for hnd in handles:
        hnd.wait()

    for buf, out in enumerate((ph_out, pt_out)):
        pltpu.sync_copy(row_bufs.at[buf], out.at[sl])


_SC_MESH = dict(core_axis_name="c", subcore_axis_name="s")


def _sc_gather_a(h, r, t, ent_emb_rm, rel_emb, rel_proj):
    vec = jax.ShapeDtypeStruct((B, D), jnp.float32)
    fn = functools.partial(
        pl.kernel,
        mesh=plsc.VectorSubcoreMesh(**_SC_MESH),
        out_type=(vec,) * 4,
        scratch_types=[
            pltpu.VMEM((3, _BPW), jnp.int32),
            pltpu.VMEM((4, _BPW, D), jnp.float32),
            pltpu.SemaphoreType.DMA,
        ],
    )(_gather_body_a)
    return fn(h, r, t, ent_emb_rm, rel_emb, rel_proj)


def _sc_gather_b(h, t, ent_proj_rm):
    vec = jax.ShapeDtypeStruct((B, D), jnp.float32)
    fn = functools.partial(
        pl.kernel,
        mesh=plsc.VectorSubcoreMesh(**_SC_MESH),
        out_type=(vec,) * 2,
        scratch_types=[
            pltpu.VMEM((2, _BPW), jnp.int32),
            pltpu.VMEM((2, _BPW, D), jnp.float32),
            pltpu.SemaphoreType.DMA,
        ],
    )(_gather_body_b)
    return fn(h, t, ent_proj_rm)


def _score_body(he_ref, re_ref, te_ref, ph_ref, pr_ref, pt_ref, out_ref):
    he = he_ref[...]
    re_ = re_ref[...]
    te = te_ref[...]
    ph = ph_ref[...]
    pr = pr_ref[...]
    pt = pt_ref[...]
    E = he + re_ - te

    def dot_t(x, y):
        return lax.dot_general(x, y, (((1,), (1,)), ((), ())),
                               preferred_element_type=jnp.float32)

    A = dot_t(he, ph)
    Bm = dot_t(re_, pr)
    C = dot_t(te, pt)
    EPh = dot_t(E, ph)
    EPr = dot_t(E, pr)
    EPt = dot_t(E, pt)

    nh = jnp.sum(ph * ph, axis=1)[None, :]
    nr = jnp.sum(pr * pr, axis=1)[None, :]
    nt = jnp.sum(pt * pt, axis=1)[None, :]
    php_r = jnp.sum(ph * pr, axis=1)[None, :]
    php_t = jnp.sum(ph * pt, axis=1)[None, :]
    prp_t = jnp.sum(pr * pt, axis=1)[None, :]
    nE = jnp.sum(E * E, axis=1)[:, None]

    s2 = (nE + A * A * nh + Bm * Bm * nr + C * C * nt
          + 2.0 * (A * EPh + Bm * EPr - C * EPt)
          + 2.0 * (A * Bm * php_r - A * C * php_t - Bm * C * prp_t))
    out_ref[...] = jnp.sqrt(jnp.maximum(s2, 0.0))


def _tc_score(he, re_, te, ph, pr, pt):
    return pl.pallas_call(
        _score_body,
        out_shape=jax.ShapeDtypeStruct((B, B), jnp.float32),
    )(he, re_, te, ph, pr, pt)


def kernel(h, r, t, ent_emb, rel_emb, ent_proj, rel_proj, batch_type):
    h = h.astype(jnp.int32)
    r = r.astype(jnp.int32)
    t = t.astype(jnp.int32)
    ent_emb_rm = _pack(ent_emb.T)
    he, te, re_, pr = _sc_gather_a(h, r, t, ent_emb_rm, rel_emb, rel_proj)
    ent_proj_rm = _pack(ent_proj.T)
    ph, pt = _sc_gather_b(h, t, ent_proj_rm)
    return _tc_score(he, re_, te, ph, pr, pt)


# speedup vs baseline: 1.0189x; 1.0189x over previous
import functools

import jax
import jax.numpy as jnp
from jax import lax
from jax.experimental import pallas as pl
from jax.experimental.pallas import tpu as pltpu
from jax.experimental.pallas import tpu_sc as plsc

NE, NR, D, B = 100000, 1000, 64, 512

_NC, _NS = 2, 16
_NW = _NC * _NS
_BPW = B // _NW
_PCH = 8192


def _pack_body(in_ref, out_ref):
    t = in_ref[...].T
    out_ref[...] = jnp.concatenate([t, jnp.zeros_like(t)], axis=1)


def _pack(table_t):
    n = table_t.shape[1]
    grid = (n + _PCH - 1) // _PCH
    return pl.pallas_call(
        _pack_body,
        grid=(grid,),
        in_specs=[pl.BlockSpec((D, _PCH), lambda j: (0, j))],
        out_specs=pl.BlockSpec((_PCH, 2 * D), lambda j: (j, 0)),
        out_shape=jax.ShapeDtypeStruct((n, 2 * D), jnp.float32),
    )(table_t)


def _gather_body_a(h_hbm, r_hbm, t_hbm, ent_emb, rel_emb, rel_proj,
                   he_out, te_out, re_out, pr_out,
                   idx_v, ebufs, rbufs, sem):
    wid = lax.axis_index("s") * _NC + lax.axis_index("c")
    base = wid * _BPW
    sl = pl.ds(base, _BPW)

    pltpu.sync_copy(h_hbm.at[sl], idx_v.at[0])
    pltpu.sync_copy(r_hbm.at[sl], idx_v.at[1])
    pltpu.sync_copy(t_hbm.at[sl], idx_v.at[2])
    hv = idx_v[0]
    rv = idx_v[1]
    tv = idx_v[2]

    handles = []
    for vec, buf in ((hv, 0), (tv, 1)):
        for k in range(_BPW):
            handles.append(pltpu.async_copy(
                ent_emb.at[pl.ds(vec[k], 1)], ebufs.at[buf, pl.ds(k, 1)], sem))
    for table, buf in ((rel_emb, 0), (rel_proj, 1)):
        for k in range(_BPW):
            handles.append(pltpu.async_copy(
                table.at[pl.ds(rv[k], 1)], rbufs.at[buf, pl.ds(k, 1)], sem))
    for hnd in handles:
        hnd.wait()

    pltpu.sync_copy(ebufs.at[0], he_out.at[sl])
    pltpu.sync_copy(ebufs.at[1], te_out.at[sl])
    pltpu.sync_copy(rbufs.at[0], re_out.at[sl])
    pltpu.sync_copy(rbufs.at[1], pr_out.at[sl])


def _gather_body_b(h_hbm, t_hbm, ent_proj,
                   ph_out, pt_out,
                   idx_v, ebufs, sem):
    wid = lax.axis_index("s") * _NC + lax.axis_index("c")
    base = wid * _BPW
    sl = pl.ds(base, _BPW)

    pltpu.sync_copy(h_hbm.at[sl], idx_v.at[0])
    pltpu.sync_copy(t_hbm.at[sl], idx_v.at[1])
    hv = idx_v[0]
    tv = idx_v[1]

    handles = []
    for vec, buf in ((hv, 0), (tv, 1)):
        for k in range(_BPW):
            handles.append(pltpu.async_copy(
                ent_proj.at[pl.ds(vec[k], 1)], ebufs.at[buf, pl.ds(k, 1)], sem))
    for hnd in handles:
        hnd.wait()

    pltpu.sync_copy(ebufs.at[0], ph_out.at[sl])
    pltpu.sync_copy(ebufs.at[1], pt_out.at[sl])


_SC_MESH = dict(core_axis_name="c", subcore_axis_name="s")


def _sc_gather_a(h, r, t, ent_emb_rm, rel_emb, rel_proj):
    wide = jax.ShapeDtypeStruct((B, 2 * D), jnp.float32)
    narrow = jax.ShapeDtypeStruct((B, D), jnp.float32)
    fn = functools.partial(
        pl.kernel,
        mesh=plsc.VectorSubcoreMesh(**_SC_MESH),
        out_type=(wide, wide, narrow, narrow),
        scratch_types=[
            pltpu.VMEM((3, _BPW), jnp.int32),
            pltpu.VMEM((2, _BPW, 2 * D), jnp.float32),
            pltpu.VMEM((2, _BPW, D), jnp.float32),
            pltpu.SemaphoreType.DMA,
        ],
    )(_gather_body_a)
    return fn(h, r, t, ent_emb_rm, rel_emb, rel_proj)


def _sc_gather_b(h, t, ent_proj_rm):
    wide = jax.ShapeDtypeStruct((B, 2 * D), jnp.float32)
    fn = functools.partial(
        pl.kernel,
        mesh=plsc.VectorSubcoreMesh(**_SC_MESH),
        out_type=(wide, wide),
        scratch_types=[
            pltpu.VMEM((2, _BPW), jnp.int32),
            pltpu.VMEM((2, _BPW, 2 * D), jnp.float32),
            pltpu.SemaphoreType.DMA,
        ],
    )(_gather_body_b)
    return fn(h, t, ent_proj_rm)


def _score_body(he_ref, re_ref, te_ref, ph_ref, pr_ref, pt_ref, out_ref):
    he = he_ref[:, :D]
    re_ = re_ref[...]
    te = te_ref[:, :D]
    ph = ph_ref[:, :D]
    pr = pr_ref[...]
    pt = pt_ref[:, :D]
    E = he + re_ - te

    def dot_t(x, y):
        return lax.dot_general(x, y, (((1,), (1,)), ((), ())),
                               preferred_element_type=jnp.float32)

    A = dot_t(he, ph)
    Bm = dot_t(re_, pr)
    C = dot_t(te, pt)
    EPh = dot_t(E, ph)
    EPr = dot_t(E, pr)
    EPt = dot_t(E, pt)

    nh = jnp.sum(ph * ph, axis=1)[None, :]
    nr = jnp.sum(pr * pr, axis=1)[None, :]
    nt = jnp.sum(pt * pt, axis=1)[None, :]
    php_r = jnp.sum(ph * pr, axis=1)[None, :]
    php_t = jnp.sum(ph * pt, axis=1)[None, :]
    prp_t = jnp.sum(pr * pt, axis=1)[None, :]
    nE = jnp.sum(E * E, axis=1)[:, None]

    s2 = (nE + A * A * nh + Bm * Bm * nr + C * C * nt
          + 2.0 * (A * EPh + Bm * EPr - C * EPt)
          + 2.0 * (A * Bm * php_r - A * C * php_t - Bm * C * prp_t))
    out_ref[...] = jnp.sqrt(jnp.maximum(s2, 0.0))


def _tc_score(he, re_, te, ph, pr, pt):
    return pl.pallas_call(
        _score_body,
        out_shape=jax.ShapeDtypeStruct((B, B), jnp.float32),
    )(he, re_, te, ph, pr, pt)


def kernel(h, r, t, ent_emb, rel_emb, ent_proj, rel_proj, batch_type):
    h = h.astype(jnp.int32)
    r = r.astype(jnp.int32)
    t = t.astype(jnp.int32)
    ent_emb_rm = _pack(ent_emb.T)
    he, te, re_, pr = _sc_gather_a(h, r, t, ent_emb_rm, rel_emb, rel_proj)
    ent_proj_rm = _pack(ent_proj.T)
    ph, pt = _sc_gather_b(h, t, ent_proj_rm)
    return _tc_score(he, re_, te, ph, pr, pt)


# speedup vs baseline: 1.0251x; 1.0061x over previous
import functools

import jax
import jax.numpy as jnp
from jax import lax
from jax.experimental import pallas as pl
from jax.experimental.pallas import tpu as pltpu
from jax.experimental.pallas import tpu_sc as plsc

NE, NR, D, B = 100000, 1000, 64, 512

_NC, _NS = 2, 16
_NW = _NC * _NS
_BPW = B // _NW
_PCH = 8192


_PSTEPS = tuple((j * _PCH, _PCH) for j in range(NE // _PCH)) + \
    (((NE // _PCH) * _PCH, (NE % _PCH) // 128 * 128),)
_PTAIL_OFF = _PSTEPS[-1][0] + _PSTEPS[-1][1]
_PTAIL_W = NE - _PTAIL_OFF


def _pack_body(in_hbm, tail_ref, out_hbm, in_v, out_v, in_sems, out_sems):
    def in_copy(j):
        off, w = _PSTEPS[j]
        return pltpu.make_async_copy(
            in_hbm.at[:, pl.ds(off, w)], in_v.at[j % 2, :, pl.ds(0, w)],
            in_sems.at[j % 2])

    def out_copy(j):
        off, w = _PSTEPS[j]
        return pltpu.make_async_copy(
            out_v.at[j % 2, pl.ds(0, w), :], out_hbm.at[pl.ds(off, w)],
            out_sems.at[j % 2])

    n = len(_PSTEPS)
    in_copy(0).start()
    for j in range(n):
        if j + 1 < n:
            in_copy(j + 1).start()
        in_copy(j).wait()
        off, w = _PSTEPS[j]
        if j >= 2:
            out_copy(j - 2).wait()
        t = in_v[j % 2, :, pl.ds(0, w)].T
        out_v[j % 2, pl.ds(0, w), :] = jnp.concatenate(
            [t, jnp.zeros_like(t)], axis=1)
        out_copy(j).start()
    out_copy(n - 2).wait()
    out_copy(n - 1).wait()

    tt = tail_ref[...].T
    out_v[0, pl.ds(0, _PTAIL_W), :] = jnp.concatenate(
        [tt, jnp.zeros_like(tt)], axis=1)
    tail_cp = pltpu.make_async_copy(
        out_v.at[0, pl.ds(0, _PTAIL_W), :], out_hbm.at[pl.ds(_PTAIL_OFF, _PTAIL_W)],
        out_sems.at[0])
    tail_cp.start()
    tail_cp.wait()


def _pack(table_t):
    n = table_t.shape[1]
    return pl.pallas_call(
        _pack_body,
        in_specs=[pl.BlockSpec(memory_space=pltpu.MemorySpace.HBM),
                  pl.BlockSpec((D, _PTAIL_W), lambda: (0, 0))],
        out_specs=pl.BlockSpec(memory_space=pltpu.MemorySpace.HBM),
        out_shape=jax.ShapeDtypeStruct((n, 2 * D), jnp.float32),
        scratch_shapes=[
            pltpu.VMEM((2, D, _PCH), jnp.float32),
            pltpu.VMEM((2, _PCH, 2 * D), jnp.float32),
            pltpu.SemaphoreType.DMA((2,)),
            pltpu.SemaphoreType.DMA((2,)),
        ],
    )(table_t, table_t[:, _PTAIL_OFF:])


def _gather_body_a(h_hbm, r_hbm, t_hbm, ent_emb, rel_emb, rel_proj,
                   he_out, te_out, re_out, pr_out,
                   idx_v, ebufs, rbufs, sem):
    wid = lax.axis_index("s") * _NC + lax.axis_index("c")
    base = wid * _BPW
    sl = pl.ds(base, _BPW)

    pltpu.sync_copy(h_hbm.at[sl], idx_v.at[0])
    pltpu.sync_copy(r_hbm.at[sl], idx_v.at[1])
    pltpu.sync_copy(t_hbm.at[sl], idx_v.at[2])
    hv = idx_v[0]
    rv = idx_v[1]
    tv = idx_v[2]

    handles = []
    for vec, buf in ((hv, 0), (tv, 1)):
        for k in range(_BPW):
            handles.append(pltpu.async_copy(
                ent_emb.at[pl.ds(vec[k], 1)], ebufs.at[buf, pl.ds(k, 1)], sem))
    for table, buf in ((rel_emb, 0), (rel_proj, 1)):
        for k in range(_BPW):
            handles.append(pltpu.async_copy(
                table.at[pl.ds(rv[k], 1)], rbufs.at[buf, pl.ds(k, 1)], sem))
    for hnd in handles:
        hnd.wait()

    pltpu.sync_copy(ebufs.at[0], he_out.at[sl])
    pltpu.sync_copy(ebufs.at[1], te_out.at[sl])
    pltpu.sync_copy(rbufs.at[0], re_out.at[sl])
    pltpu.sync_copy(rbufs.at[1], pr_out.at[sl])


def _gather_body_b(h_hbm, t_hbm, ent_proj,
                   ph_out, pt_out,
                   idx_v, ebufs, sem):
    wid = lax.axis_index("s") * _NC + lax.axis_index("c")
    base = wid * _BPW
    sl = pl.ds(base, _BPW)

    pltpu.sync_copy(h_hbm.at[sl], idx_v.at[0])
    pltpu.sync_copy(t_hbm.at[sl], idx_v.at[1])
    hv = idx_v[0]
    tv = idx_v[1]

    handles = []
    for vec, buf in ((hv, 0), (tv, 1)):
        for k in range(_BPW):
            handles.append(pltpu.async_copy(
                ent_proj.at[pl.ds(vec[k], 1)], ebufs.at[buf, pl.ds(k, 1)], sem))
    for hnd in handles:
        hnd.wait()

    pltpu.sync_copy(ebufs.at[0], ph_out.at[sl])
    pltpu.sync_copy(ebufs.at[1], pt_out.at[sl])


_SC_MESH = dict(core_axis_name="c", subcore_axis_name="s")


def _sc_gather_a(h, r, t, ent_emb_rm, rel_emb, rel_proj):
    wide = jax.ShapeDtypeStruct((B, 2 * D), jnp.float32)
    narrow = jax.ShapeDtypeStruct((B, D), jnp.float32)
    fn = functools.partial(
        pl.kernel,
        mesh=plsc.VectorSubcoreMesh(**_SC_MESH),
        out_type=(wide, wide, narrow, narrow),
        scratch_types=[
            pltpu.VMEM((3, _BPW), jnp.int32),
            pltpu.VMEM((2, _BPW, 2 * D), jnp.float32),
            pltpu.VMEM((2, _BPW, D), jnp.float32),
            pltpu.SemaphoreType.DMA,
        ],
    )(_gather_body_a)
    return fn(h, r, t, ent_emb_rm, rel_emb, rel_proj)


def _sc_gather_b(h, t, ent_proj_rm):
    wide = jax.ShapeDtypeStruct((B, 2 * D), jnp.float32)
    fn = functools.partial(
        pl.kernel,
        mesh=plsc.VectorSubcoreMesh(**_SC_MESH),
        out_type=(wide, wide),
        scratch_types=[
            pltpu.VMEM((2, _BPW), jnp.int32),
            pltpu.VMEM((2, _BPW, 2 * D), jnp.float32),
            pltpu.SemaphoreType.DMA,
        ],
    )(_gather_body_b)
    return fn(h, t, ent_proj_rm)


def _score_body(he_ref, re_ref, te_ref, ph_ref, pr_ref, pt_ref, out_ref):
    he = he_ref[:, :D]
    re_ = re_ref[...]
    te = te_ref[:, :D]
    ph = ph_ref[:, :D]
    pr = pr_ref[...]
    pt = pt_ref[:, :D]
    E = he + re_ - te

    def dot_t(x, y):
        return lax.dot_general(x, y, (((1,), (1,)), ((), ())),
                               preferred_element_type=jnp.float32)

    A = dot_t(he, ph)
    Bm = dot_t(re_, pr)
    C = dot_t(te, pt)
    EPh = dot_t(E, ph)
    EPr = dot_t(E, pr)
    EPt = dot_t(E, pt)

    nh = jnp.sum(ph * ph, axis=1)[None, :]
    nr = jnp.sum(pr * pr, axis=1)[None, :]
    nt = jnp.sum(pt * pt, axis=1)[None, :]
    php_r = jnp.sum(ph * pr, axis=1)[None, :]
    php_t = jnp.sum(ph * pt, axis=1)[None, :]
    prp_t = jnp.sum(pr * pt, axis=1)[None, :]
    nE = jnp.sum(E * E, axis=1)[:, None]

    s2 = (nE + A * A * nh + Bm * Bm * nr + C * C * nt
          + 2.0 * (A * EPh + Bm * EPr - C * EPt)
          + 2.0 * (A * Bm * php_r - A * C * php_t - Bm * C * prp_t))
    out_ref[...] = jnp.sqrt(jnp.maximum(s2, 0.0))


def _tc_score(he, re_, te, ph, pr, pt):
    return pl.pallas_call(
        _score_body,
        out_shape=jax.ShapeDtypeStruct((B, B), jnp.float32),
    )(he, re_, te, ph, pr, pt)


def kernel(h, r, t, ent_emb, rel_emb, ent_proj, rel_proj, batch_type):
    h = h.astype(jnp.int32)
    r = r.astype(jnp.int32)
    t = t.astype(jnp.int32)
    ent_emb_rm = _pack(ent_emb.T)
    he, te, re_, pr = _sc_gather_a(h, r, t, ent_emb_rm, rel_emb, rel_proj)
    ent_proj_rm = _pack(ent_proj.T)
    ph, pt = _sc_gather_b(h, t, ent_proj_rm)
    return _tc_score(he, re_, te, ph, pr, pt)
